# baseline (speedup 1.0000x reference)
import functools

import jax
import jax.numpy as jnp
from jax import lax
from jax.experimental import pallas as pl
from jax.experimental.pallas import tpu as pltpu
from jax.experimental.pallas import tpu_sc as plsc

NUM_FIELDS = 26
VOCAB = 100000
DIM = 64
BATCH = 16384

_info = plsc.get_sparse_core_info()
_NC, _NS, _L = _info.num_cores, _info.num_subcores, _info.num_lanes
_NW = _NC * _NS

_BC = 256
_FULLB = VOCAB // _BC
_MEDC0 = _FULLB * _BC
_TAILC0 = _MEDC0 + 128
_NBLK = NUM_FIELDS * _FULLB
_BPWK = 318
_NPAIR = NUM_FIELDS * VOCAB // 2

_BPW = BATCH // _NW
_IPW = _BPW * NUM_FIELDS
_CB = 4
_ROWS = _CB * NUM_FIELDS
_NCHUNK = _BPW // _CB

_mesh = plsc.VectorSubcoreMesh(core_axis_name="c", subcore_axis_name="s")
_tiled_params = pltpu.CompilerParams(
    use_tc_tiling_on_sc=True, needs_layout_passes=False
)
_lane = None


def _transpose_block(src_v, dst_v, ncols, lane):
    @plsc.parallel_loop(0, ncols, unroll=16)
    def _cols(v):
        dst_r = lax.shift_right_logical(v, 1)
        dst_c0 = (v & 1) * DIM
        col = lax.broadcast(v, (_L,))
        for j in range(DIM // _L):
            vals = plsc.load_gather(src_v, [j * _L + lane, col])
            dst_v[dst_r, pl.ds(dst_c0 + j * _L, _L)] = vals


@functools.partial(
    pl.kernel,
    mesh=_mesh,
    compiler_params=_tiled_params,
    out_type=jax.ShapeDtypeStruct((_NPAIR, 128), jnp.float32),
    scratch_types=[
        pltpu.VMEM((DIM, _BC), jnp.float32),
        pltpu.VMEM((DIM, _BC), jnp.float32),
        pltpu.VMEM((_BC // 2, 128), jnp.float32),
        pltpu.VMEM((_BC // 2, 128), jnp.float32),
        pltpu.VMEM((DIM, 128), jnp.float32),
        pltpu.VMEM((16, 128), jnp.float32),
        pltpu.SemaphoreType.DMA,
        pltpu.SemaphoreType.DMA,
        pltpu.SemaphoreType.DMA,
        pltpu.SemaphoreType.DMA,
    ],
)
def _relayout(
    tab_hbm, tail_hbm, out_hbm, in0, in1, tb0, tb1, med_v, tail_v,
    si0, si1, so0, so1,
):
    wid = lax.axis_index("s") * _NC + lax.axis_index("c")
    lane = lax.iota(jnp.int32, _L)
    inb, tbb, sib, sob = (in0, in1), (tb0, tb1), (si0, si1), (so0, so1)

    def _src_slice(b):
        breal = jnp.where(b < _NBLK, b, 0)
        f = breal // _FULLB
        cb = breal % _FULLB
        return tab_hbm.at[
            pl.ds(pl.multiple_of(f * DIM, DIM), DIM),
            pl.ds(pl.multiple_of(cb * _BC, _BC), _BC),
        ], f, cb

    def _dst_slice(f, cb):
        orow = f * (VOCAB // 2) + cb * (_BC // 2)
        return out_hbm.at[
            pl.ds(pl.multiple_of(orow, _BC // 2), _BC // 2), :
        ]

    for k in (0, 1):
        src, _, _ = _src_slice(wid * _BPWK + k)
        pltpu.async_copy(src, inb[k], sib[k])

    def _pair(g, _):
        for k in (0, 1):
            i = g * 2 + k
            b = wid * _BPWK + i
            src, f, cb = _src_slice(b)
            pltpu.make_async_copy(src, inb[k], sib[k]).wait()

            @pl.when(i >= 2)
            def _():
                pltpu.make_async_copy(tbb[k], _dst_slice(f, cb), sob[k]).wait()

            _transpose_block(inb[k], tbb[k], _BC, lane)
            pltpu.async_copy(tbb[k], _dst_slice(f, cb), sob[k])

            @pl.when(i + 2 < _BPWK)
            def _():
                src2, _, _ = _src_slice(b + 2)
                pltpu.async_copy(src2, inb[k], sib[k])
        return _

    lax.fori_loop(0, _BPWK // 2, _pair, None)
    for k in (0, 1):
        pltpu.make_async_copy(tbb[k], _dst_slice(0, 0), sob[k]).wait()

    @pl.when(wid < NUM_FIELDS)
    def _():
        f = wid
        pltpu.sync_copy(
            tab_hbm.at[
                pl.ds(pl.multiple_of(f * DIM, DIM), DIM),
                pl.ds(_MEDC0, 128),
            ],
            med_v,
        )
        _transpose_block(med_v, tb0, 128, lane)
        pltpu.sync_copy(
            tb0.at[pl.ds(0, 64), :],
            out_hbm.at[
                pl.ds(pl.multiple_of(f * (VOCAB // 2) + _MEDC0 // 2, 8), 64), :
            ],
        )
        pltpu.sync_copy(
            tail_hbm.at[pl.ds(pl.multiple_of(f * 16, 16), 16), :], tail_v
        )
        pltpu.sync_copy(
            tail_v,
            out_hbm.at[
                pl.ds(pl.multiple_of(f * (VOCAB // 2) + _TAILC0 // 2, 8), 16), :
            ],
        )


@functools.partial(
    pl.kernel,
    mesh=_mesh,
    compiler_params=_tiled_params,
    out_type=jax.ShapeDtypeStruct((BATCH // 2, 128), jnp.float32),
    scratch_types=[
        pltpu.VMEM((_IPW,), jnp.int32),
        pltpu.VMEM((_IPW,), jnp.int32),
        pltpu.VMEM((_ROWS, 128), jnp.float32),
        pltpu.VMEM((_ROWS, 128), jnp.float32),
        pltpu.VMEM((_BPW // 2, 128), jnp.float32),
        pltpu.SemaphoreType.DMA,
        pltpu.SemaphoreType.DMA,
    ],
)
def _emb_sum(idx_hbm, tab_hbm, out_hbm, idx_v, hoff_v, r0, r1, out_v, sg0, sg1):
    wid = lax.axis_index("s") * _NC + lax.axis_index("c")
    base_b = wid * _BPW
    lane = lax.iota(jnp.int32, _L)
    rows, sgs = (r0, r1), (sg0, sg1)

    pltpu.sync_copy(
        idx_hbm.at[pl.ds(pl.multiple_of(base_b * NUM_FIELDS, _IPW), _IPW)], idx_v
    )

    def _convert(i, _):
        off = i * _L
        pos = lane + off
        field = lax.rem(pos, NUM_FIELDS)
        g = idx_v[pl.ds(off, _L)] + field * VOCAB
        idx_v[pl.ds(off, _L)] = lax.shift_right_logical(g, 1)
        hoff_v[pl.ds(off, _L)] = (g & 1) * DIM
        return _

    lax.fori_loop(0, _IPW // _L, _convert, None)

    def _gather_src(c):
        return tab_hbm.at[idx_v.at[pl.ds(c * _ROWS, _ROWS)]]

    for k in (0, 1):
        pltpu.async_copy(_gather_src(k), rows[k], sgs[k])

    def _chunk_pair(g, _):
        for k in (0, 1):
            c = g * 2 + k
            rv = rows[k]
            pltpu.make_async_copy(_gather_src(c), rv, sgs[k]).wait()
            hv = [hoff_v[pl.ds(c * _ROWS + t * 16, 16)] for t in range(6)]
            hv.append(hoff_v[pl.ds(c * _ROWS + _ROWS - 16, 16)])
            for b in range(_CB):
                accs = [None] * (DIM // _L)
                for f in range(NUM_FIELDS):
                    r = b * NUM_FIELDS + f
                    li, pos = (
                        (6, r - (_ROWS - 16)) if r >= _ROWS - 16
                        else (r // 16, r % 16)
                    )
                    cbase = jnp.take_along_axis(
                        hv[li], jnp.full((_L,), pos, jnp.int32), axis=0
                    )
                    row = jnp.full((_L,), r, jnp.int32)
                    for j in range(DIM // _L):
                        vals = plsc.load_gather(
                            rv, [row, cbase + (j * _L + lane)]
                        )
                        accs[j] = vals if accs[j] is None else accs[j] + vals
                orow = c * (_CB // 2) + b // 2
                oc0 = (b % 2) * DIM
                for j in range(DIM // _L):
                    out_v[orow, pl.ds(oc0 + j * _L, _L)] = accs[j]

            @pl.when(c + 2 < _NCHUNK)
            def _():
                pltpu.async_copy(_gather_src(c + 2), rv, sgs[k])
        return _

    lax.fori_loop(0, _NCHUNK // 2, _chunk_pair, None)

    pltpu.sync_copy(
        out_v,
        out_hbm.at[pl.ds(pl.multiple_of(base_b // 2, _BPW // 2), _BPW // 2), :],
    )


def kernel(inputs, tables):
    tab_t = jnp.transpose(tables, (0, 2, 1)).reshape(NUM_FIELDS * DIM, VOCAB)
    tail = lax.slice(
        tables, (0, _TAILC0, 0), (NUM_FIELDS, VOCAB, DIM)
    ).reshape(NUM_FIELDS * (VOCAB - _TAILC0) // 2, 128)
    pairs = _relayout(tab_t, tail)
    flat_idx = inputs.reshape(-1)
    out = _emb_sum(flat_idx, pairs)
    return out.reshape(BATCH, DIM)


# speedup vs baseline: 3.2567x; 3.2567x over previous
import functools

import jax
import jax.numpy as jnp
from jax import lax
from jax.experimental import pallas as pl
from jax.experimental.pallas import tpu as pltpu
from jax.experimental.pallas import tpu_sc as plsc

NUM_FIELDS = 26
VOCAB = 100000
DIM = 64
BATCH = 16384

_info = plsc.get_sparse_core_info()
_NC, _NS, _L = _info.num_cores, _info.num_subcores, _info.num_lanes
_NW = _NC * _NS

_BC = 256
_FULLB = VOCAB // _BC
_MEDC0 = _FULLB * _BC
_TAILC0 = _MEDC0 + 128
_NBLK = NUM_FIELDS * _FULLB
_BPWK = 318
_NPAIR = NUM_FIELDS * VOCAB // 2

_BPW = BATCH // _NW
_IPW = _BPW * NUM_FIELDS
_CB = 4
_ROWS = _CB * NUM_FIELDS
_NCHUNK = _BPW // _CB

_mesh = plsc.VectorSubcoreMesh(core_axis_name="c", subcore_axis_name="s")
_tiled_params = pltpu.CompilerParams(
    use_tc_tiling_on_sc=True, needs_layout_passes=False
)
_lane = None


def _transpose_block(src_v, dst1, ncols, lane):
    rot = [lax.rem(lane + k, _L) for k in range(_L)]
    dcon = [
        lax.shift_right_logical(rot[k], 1) * 128 + (rot[k] & 1) * DIM + lane
        for k in range(_L)
    ]
    nvg = ncols // _L

    @plsc.parallel_loop(0, (DIM // _L) * nvg, unroll=2)
    def _subtile(t):
        d0 = (t // nvg) * _L
        v0 = lax.rem(t, nvg) * _L
        row = lax.broadcast(d0, (_L,)) + lane
        vb = lax.broadcast(v0, (_L,))
        db = lax.broadcast(lax.shift_right_logical(v0, 1) * 128 + d0, (_L,))
        for k in range(_L):
            vals = plsc.load_gather(src_v, [row, vb + rot[k]])
            plsc.store_scatter(dst1, [db + dcon[k]], vals)


@functools.partial(
    pl.kernel,
    mesh=_mesh,
    compiler_params=_tiled_params,
    out_type=jax.ShapeDtypeStruct((_NPAIR * 128,), jnp.float32),
    scratch_types=[
        pltpu.VMEM((DIM, _BC), jnp.float32),
        pltpu.VMEM((DIM, _BC), jnp.float32),
        pltpu.VMEM(((_BC // 2) * 128,), jnp.float32),
        pltpu.VMEM(((_BC // 2) * 128,), jnp.float32),
        pltpu.VMEM((DIM, 128), jnp.float32),
        pltpu.VMEM((2048,), jnp.float32),
        pltpu.SemaphoreType.DMA,
        pltpu.SemaphoreType.DMA,
        pltpu.SemaphoreType.DMA,
        pltpu.SemaphoreType.DMA,
    ],
)
def _relayout(
    tab_hbm, tail_hbm, out_hbm, in0, in1, tb0, tb1, med_v, tail_v,
    si0, si1, so0, so1,
):
    wid = lax.axis_index("s") * _NC + lax.axis_index("c")
    lane = lax.iota(jnp.int32, _L)
    inb, tbb, sib, sob = (in0, in1), (tb0, tb1), (si0, si1), (so0, so1)

    def _src_slice(b):
        breal = jnp.where(b < _NBLK, b, 0)
        f = breal // _FULLB
        cb = breal % _FULLB
        return tab_hbm.at[
            pl.ds(pl.multiple_of(f * DIM, DIM), DIM),
            pl.ds(pl.multiple_of(cb * _BC, _BC), _BC),
        ], f, cb

    def _dst_slice(f, cb):
        oword = (f * (VOCAB // 2) + cb * (_BC // 2)) * 128
        return out_hbm.at[
            pl.ds(pl.multiple_of(oword, (_BC // 2) * 128), (_BC // 2) * 128)
        ]

    for k in (0, 1):
        src, _, _ = _src_slice(wid * _BPWK + k)
        pltpu.async_copy(src, inb[k], sib[k])

    def _pair(g, _):
        for k in (0, 1):
            i = g * 2 + k
            b = wid * _BPWK + i
            src, f, cb = _src_slice(b)
            pltpu.make_async_copy(src, inb[k], sib[k]).wait()

            @pl.when(i >= 2)
            def _():
                pltpu.make_async_copy(tbb[k], _dst_slice(f, cb), sob[k]).wait()

            _transpose_block(inb[k], tbb[k], _BC, lane)
            pltpu.async_copy(tbb[k], _dst_slice(f, cb), sob[k])

            @pl.when(i + 2 < _BPWK)
            def _():
                src2, _, _ = _src_slice(b + 2)
                pltpu.async_copy(src2, inb[k], sib[k])
        return _

    lax.fori_loop(0, _BPWK // 2, _pair, None)
    for k in (0, 1):
        pltpu.make_async_copy(tbb[k], _dst_slice(0, 0), sob[k]).wait()

    @pl.when(wid < NUM_FIELDS)
    def _():
        f = wid
        pltpu.sync_copy(
            tab_hbm.at[
                pl.ds(pl.multiple_of(f * DIM, DIM), DIM),
                pl.ds(_MEDC0, 128),
            ],
            med_v,
        )
        _transpose_block(med_v, tb0.at[pl.ds(0, 64 * 128)], 128, lane)
        pltpu.sync_copy(
            tb0.at[pl.ds(0, 64 * 128)],
            out_hbm.at[
                pl.ds(
                    pl.multiple_of(
                        (f * (VOCAB // 2) + _MEDC0 // 2) * 128, 128
                    ),
                    64 * 128,
                )
            ],
        )
        pltpu.sync_copy(
            tail_hbm.at[pl.ds(pl.multiple_of(f * 2048, 2048), 2048)], tail_v
        )
        pltpu.sync_copy(
            tail_v,
            out_hbm.at[
                pl.ds(
                    pl.multiple_of(
                        (f * (VOCAB // 2) + _TAILC0 // 2) * 128, 128
                    ),
                    2048,
                )
            ],
        )


@functools.partial(
    pl.kernel,
    mesh=_mesh,
    compiler_params=_tiled_params,
    out_type=jax.ShapeDtypeStruct((BATCH // 2, 128), jnp.float32),
    scratch_types=[
        pltpu.VMEM((_IPW,), jnp.int32),
        pltpu.VMEM((_IPW,), jnp.int32),
        pltpu.VMEM((_ROWS, 128), jnp.float32),
        pltpu.VMEM((_ROWS, 128), jnp.float32),
        pltpu.VMEM((_BPW // 2, 128), jnp.float32),
        pltpu.SemaphoreType.DMA,
        pltpu.SemaphoreType.DMA,
    ],
)
def _emb_sum(idx_hbm, tab_hbm, out_hbm, idx_v, hoff_v, r0, r1, out_v, sg0, sg1):
    wid = lax.axis_index("s") * _NC + lax.axis_index("c")
    base_b = wid * _BPW
    lane = lax.iota(jnp.int32, _L)
    rows, sgs = (r0, r1), (sg0, sg1)

    pltpu.sync_copy(
        idx_hbm.at[pl.ds(pl.multiple_of(base_b * NUM_FIELDS, _IPW), _IPW)], idx_v
    )

    def _convert(i, _):
        off = i * _L
        pos = lane + off
        field = lax.rem(pos, NUM_FIELDS)
        g = idx_v[pl.ds(off, _L)] + field * VOCAB
        idx_v[pl.ds(off, _L)] = lax.shift_right_logical(g, 1)
        hoff_v[pl.ds(off, _L)] = (g & 1) * DIM
        return _

    lax.fori_loop(0, _IPW // _L, _convert, None)

    def _gather_src(c):
        return tab_hbm.at[idx_v.at[pl.ds(c * _ROWS, _ROWS)]]

    for k in (0, 1):
        pltpu.async_copy(_gather_src(k), rows[k], sgs[k])

    def _chunk_pair(g, _):
        for k in (0, 1):
            c = g * 2 + k
            rv = rows[k]
            pltpu.make_async_copy(_gather_src(c), rv, sgs[k]).wait()
            hv = [hoff_v[pl.ds(c * _ROWS + t * 16, 16)] for t in range(6)]
            hv.append(hoff_v[pl.ds(c * _ROWS + _ROWS - 16, 16)])
            for b in range(_CB):
                accs = [None] * (DIM // _L)
                for f in range(NUM_FIELDS):
                    r = b * NUM_FIELDS + f
                    li, pos = (
                        (6, r - (_ROWS - 16)) if r >= _ROWS - 16
                        else (r // 16, r % 16)
                    )
                    cbase = jnp.take_along_axis(
                        hv[li], jnp.full((_L,), pos, jnp.int32), axis=0
                    )
                    row = jnp.full((_L,), r, jnp.int32)
                    for j in range(DIM // _L):
                        vals = plsc.load_gather(
                            rv, [row, cbase + (j * _L + lane)]
                        )
                        accs[j] = vals if accs[j] is None else accs[j] + vals
                orow = c * (_CB // 2) + b // 2
                oc0 = (b % 2) * DIM
                for j in range(DIM // _L):
                    out_v[orow, pl.ds(oc0 + j * _L, _L)] = accs[j]

            @pl.when(c + 2 < _NCHUNK)
            def _():
                pltpu.async_copy(_gather_src(c + 2), rv, sgs[k])
        return _

    lax.fori_loop(0, _NCHUNK // 2, _chunk_pair, None)

    pltpu.sync_copy(
        out_v,
        out_hbm.at[pl.ds(pl.multiple_of(base_b // 2, _BPW // 2), _BPW // 2), :],
    )


def kernel(inputs, tables):
    tab_t = jnp.transpose(tables, (0, 2, 1)).reshape(NUM_FIELDS * DIM, VOCAB)
    tail = lax.slice(
        tables, (0, _TAILC0, 0), (NUM_FIELDS, VOCAB, DIM)
    ).reshape(-1)
    pairs = _relayout(tab_t, tail).reshape(_NPAIR, 128)
    flat_idx = inputs.reshape(-1)
    out = _emb_sum(flat_idx, pairs)
    return out.reshape(BATCH, DIM)
